# baseline (speedup 1.0000x reference)
import functools
import math

import jax
import jax.numpy as jnp
from jax.experimental import pallas as pl
from jax.experimental.pallas import tpu as pltpu
from jax.experimental.pallas import tpu_sc as plsc

_N = 4096
_K = 32
_H = 128
_NG = 50
_NGP = 64
_G = 256
_CUT = 10.0
_CUT2 = _CUT * _CUT
_TB = 128
_WC = 512
_RB = 256
_PB = _RB * _K
_LOG2 = math.log(2.0)
_BIGV = 1e30
_BIGI = 1e9


def _ssp(v):
    return jnp.maximum(v, 0.0) + jnp.log1p(jnp.exp(-jnp.abs(v))) - _LOG2


def _nbr_body(pos_ref, posT_ref, batch_ref, batchT_ref, nbr_ref, d_ref):
    i = pl.program_id(0)
    tgx = posT_ref[0:1, :]
    tgy = posT_ref[1:2, :]
    tgz = posT_ref[2:3, :]
    tb = batchT_ref[0:1, :]
    b_lo = tb[:, 0:1]
    b_hi = tb[:, _TB - 1:_TB]
    tgid = jax.lax.broadcasted_iota(jnp.int32, (1, _TB), 1).astype(jnp.float32) + (i * _TB).astype(jnp.float32)

    allb = batch_ref[...]
    colf = jax.lax.broadcasted_iota(jnp.int32, (_N, 1), 0).astype(jnp.float32)
    w0f = jnp.min(jnp.where(allb == b_lo, colf, float(_N)))
    c1f = jnp.max(jnp.where(allb == b_hi, colf, -1.0)) + 1.0
    w0 = (w0f.astype(jnp.int32) // 8) * 8
    c1 = c1f.astype(jnp.int32)
    nchunks = (c1 - w0 + _WC - 1) // _WC

    def chunk(c, carry):
        bv, bi = carry
        start0 = w0 + c * _WC
        start = jnp.minimum(start0, _N - _WC)
        cand = pos_ref[pl.ds(start, _WC), :]
        cb = batch_ref[pl.ds(start, _WC), :]
        cidx = (jax.lax.broadcasted_iota(jnp.int32, (_WC, 1), 0).astype(jnp.float32)
                + start.astype(jnp.float32))
        dx = cand[:, 0:1] - tgx
        dy = cand[:, 1:2] - tgy
        dz = cand[:, 2:3] - tgz
        d2 = dx * dx + dy * dy + dz * dz
        valid = ((cb == tb) & (cidx != tgid) & (d2 < _CUT2)
                 & (cidx >= start0.astype(jnp.float32)))
        vals = jnp.where(valid, d2, _BIGV)
        cv = jnp.concatenate([vals, bv], axis=0)
        ci = jnp.concatenate([jnp.broadcast_to(cidx, (_WC, _TB)), bi], axis=0)
        nv, ni = [], []
        for _ in range(_K):
            m = jnp.min(cv, axis=0, keepdims=True)
            sel = jnp.min(jnp.where(cv == m, ci, _BIGI), axis=0, keepdims=True)
            kill = (cv == m) & (ci == sel)
            cv = jnp.where(kill, _BIGV, cv)
            ci = jnp.where(kill, _BIGI, ci)
            nv.append(m)
            ni.append(sel)
        return jnp.concatenate(nv, axis=0), jnp.concatenate(ni, axis=0)

    bv0 = jnp.full((_K, _TB), _BIGV, jnp.float32)
    bi0 = jnp.full((_K, _TB), _BIGI, jnp.float32)
    bv, bi = jax.lax.fori_loop(0, nchunks, chunk, (bv0, bi0))
    ok = bv < _CUT2
    d_ref[...] = jnp.where(ok, jnp.sqrt(jnp.maximum(bv, 1e-12)), 1e10)
    nbr_ref[...] = jnp.where(ok, bi, 0.0).astype(jnp.int32)


def _nbr(pos, posT, batchf, batchfT):
    return pl.pallas_call(
        _nbr_body,
        grid=(_N // _TB,),
        in_specs=[
            pl.BlockSpec((_N, 3), lambda i: (0, 0)),
            pl.BlockSpec((3, _TB), lambda i: (0, i)),
            pl.BlockSpec((_N, 1), lambda i: (0, 0)),
            pl.BlockSpec((1, _TB), lambda i: (0, i)),
        ],
        out_specs=[
            pl.BlockSpec((_K, _TB), lambda i: (0, i)),
            pl.BlockSpec((_K, _TB), lambda i: (0, i)),
        ],
        out_shape=[
            jax.ShapeDtypeStruct((_K, _N), jnp.int32),
            jax.ShapeDtypeStruct((_K, _N), jnp.float32),
        ],
    )(pos, posT, batchf, batchfT)


def _embed_body(x_ref, emb_ref, o_ref):
    zf = x_ref[:, 5:6].astype(jnp.int32).astype(jnp.float32)
    oh = (zf == jax.lax.broadcasted_iota(jnp.int32, (1, _H), 1).astype(jnp.float32)).astype(jnp.float32)
    o_ref[...] = jnp.dot(oh, emb_ref[...], preferred_element_type=jnp.float32)


def _embed(x, emb_p):
    return pl.pallas_call(
        _embed_body,
        grid=(_N // _RB,),
        in_specs=[
            pl.BlockSpec((_RB, 11), lambda i: (i, 0)),
            pl.BlockSpec((_H, _H), lambda i: (0, 0)),
        ],
        out_specs=pl.BlockSpec((_RB, _H), lambda i: (i, 0)),
        out_shape=jax.ShapeDtypeStruct((_N, _H), jnp.float32),
    )(x, emb_p)


def _hx_body(h_ref, w_ref, o_ref):
    o_ref[...] = jnp.dot(h_ref[...], w_ref[...], preferred_element_type=jnp.float32)


def _hx(h, w):
    return pl.pallas_call(
        _hx_body,
        grid=(_N // _RB,),
        in_specs=[
            pl.BlockSpec((_RB, _H), lambda i: (i, 0)),
            pl.BlockSpec((_H, _H), lambda i: (0, 0)),
        ],
        out_specs=pl.BlockSpec((_RB, _H), lambda i: (i, 0)),
        out_shape=jax.ShapeDtypeStruct((_N, _H), jnp.float32),
    )(h, w)


_SC_NW = 32
_SC_BW = (_N * _K) // _SC_NW
_SC_CH = 128


def _sc_gather(table, idx):
    mesh = plsc.VectorSubcoreMesh(core_axis_name="c", subcore_axis_name="s")

    @functools.partial(
        pl.kernel,
        mesh=mesh,
        out_type=jax.ShapeDtypeStruct((_N * _K, _H), jnp.float32),
        scratch_types=[
            pltpu.VMEM((_SC_CH,), jnp.int32),
            pltpu.VMEM((_SC_CH, _H), jnp.float32),
            pltpu.SemaphoreType.DMA,
        ],
    )
    def gather_k(table_hbm, idx_hbm, out_hbm, idx_v, rows_v, sem):
        wid = jax.lax.axis_index("s") * 2 + jax.lax.axis_index("c")
        wbase = wid * _SC_BW

        def chunk(c, carry):
            base = wbase + c * _SC_CH
            pltpu.sync_copy(idx_hbm.at[pl.ds(base, _SC_CH)], idx_v)
            pltpu.async_copy(table_hbm.at[idx_v], rows_v, sem).wait()
            pltpu.sync_copy(rows_v, out_hbm.at[pl.ds(base, _SC_CH)])
            return carry

        jax.lax.fori_loop(0, _SC_BW // _SC_CH, chunk, 0)

    return gather_k(table, idx)


def _msg_body(d_ref, g_ref, h_ref, offs_ref, coeff_ref, w1_ref, b1_ref,
              w2_ref, b2_ref, l2w_ref, l2b_ref, l3w_ref, l3b_ref, o_ref):
    d = d_ref[...]
    dd = d - offs_ref[...]
    e = jnp.exp(coeff_ref[0, 0] * (dd * dd))
    f1 = _ssp(jnp.dot(e, w1_ref[...], preferred_element_type=jnp.float32) + b1_ref[...])
    wf = jnp.dot(f1, w2_ref[...], preferred_element_type=jnp.float32) + b2_ref[...]
    dc = jnp.minimum(d, _CUT)
    cc = 0.5 * (jnp.cos(dc * (math.pi / _CUT)) + 1.0) * (d < _CUT).astype(jnp.float32)
    msg = wf * cc * g_ref[...]
    agg = jnp.sum(msg.reshape(_RB, _K, _H), axis=1)
    v = jnp.dot(agg, l2w_ref[...], preferred_element_type=jnp.float32) + l2b_ref[...]
    v = _ssp(v)
    v = jnp.dot(v, l3w_ref[...], preferred_element_type=jnp.float32) + l3b_ref[...]
    o_ref[...] = h_ref[...] + v


def _msg(d_flat, g, h, offs_p, coeff, w1p, b1, w2, b2, l2w, l2b, l3w, l3b):
    full = lambda a, b: pl.BlockSpec((a, b), lambda i: (0, 0))
    return pl.pallas_call(
        _msg_body,
        grid=(_N // _RB,),
        in_specs=[
            pl.BlockSpec((_PB, 1), lambda i: (i, 0)),
            pl.BlockSpec((_PB, _H), lambda i: (i, 0)),
            pl.BlockSpec((_RB, _H), lambda i: (i, 0)),
            full(1, _NGP), full(1, 1), full(_NGP, _H), full(1, _H),
            full(_H, _H), full(1, _H), full(_H, _H), full(1, _H),
            full(_H, _H), full(1, _H),
        ],
        out_specs=pl.BlockSpec((_RB, _H), lambda i: (i, 0)),
        out_shape=jax.ShapeDtypeStruct((_N, _H), jnp.float32),
    )(d_flat, g, h, offs_p, coeff, w1p, b1, w2, b2, l2w, l2b, l3w, l3b)


def _readout_body(h_ref, w1_ref, b1_ref, w2_ref, b2_ref, batch_ref, o_ref):
    t = _ssp(jnp.dot(h_ref[...], w1_ref[...], preferred_element_type=jnp.float32)
             + b1_ref[...])
    s = jnp.sum(t * w2_ref[...], axis=1, keepdims=True) + b2_ref[0, 0]
    gio = jax.lax.broadcasted_iota(jnp.int32, (1, _G), 1).astype(jnp.float32)
    m = (batch_ref[...] == gio).astype(jnp.float32)
    o_ref[...] = jnp.sum(m * s, axis=0, keepdims=True)


def _readout(h, w1, b1, w2, b2, batchf):
    full = lambda a, b: pl.BlockSpec((a, b), lambda i: (0, 0))
    return pl.pallas_call(
        _readout_body,
        grid=(1,),
        in_specs=[full(_N, _H), full(_H, 64), full(1, 64), full(1, 64),
                  full(1, 1), full(_N, 1)],
        out_specs=full(1, _G),
        out_shape=jax.ShapeDtypeStruct((1, _G), jnp.float32),
    )(h, w1, b1, w2, b2, batchf)


def kernel(x, pos, batch, emb, mlp_w1, mlp_b1, mlp_w2, mlp_b2, lin1_w, lin2_w,
           lin2_b, lin3_w, lin3_b, out_w1, out_b1, out_w2, out_b2):
    batchf = batch.astype(jnp.float32).reshape(_N, 1)
    batchfT = batchf.reshape(1, _N)
    posT = pos.T

    nbrT, dT = _nbr(pos, posT, batchf, batchfT)
    nbr_flat = nbrT.T.reshape(_N * _K)
    d_flat = dT.T.reshape(_N * _K, 1)

    emb_p = jnp.zeros((_H, _H), jnp.float32).at[:100].set(emb)
    h = _embed(x, emb_p)

    offs = jnp.linspace(0.0, _CUT, _NG)
    coeff = (-0.5 / (offs[1] - offs[0]) ** 2).reshape(1, 1).astype(jnp.float32)
    offs_p = jnp.concatenate([offs.astype(jnp.float32),
                              jnp.full((_NGP - _NG,), 1e6, jnp.float32)]).reshape(1, _NGP)
    w1p = jnp.pad(mlp_w1, ((0, 0), (0, _NGP - _NG), (0, 0)))

    for i in range(6):
        hx = _hx(h, lin1_w[i])
        g = _sc_gather(hx, nbr_flat)
        h = _msg(d_flat, g, h, offs_p, coeff, w1p[i], mlp_b1[i].reshape(1, _H),
                 mlp_w2[i], mlp_b2[i].reshape(1, _H), lin2_w[i],
                 lin2_b[i].reshape(1, _H), lin3_w[i], lin3_b[i].reshape(1, _H))

    out = _readout(h, out_w1, out_b1.reshape(1, 64), out_w2.reshape(1, 64),
                   out_b2.reshape(1, 1), batchf)
    return out.reshape(-1)


# speedup vs baseline: 6.2819x; 6.2819x over previous
import functools
import math

import jax
import jax.numpy as jnp
from jax.experimental import pallas as pl
from jax.experimental.pallas import tpu as pltpu
from jax.experimental.pallas import tpu_sc as plsc

_N = 4096
_K = 32
_H = 128
_NG = 50
_NGP = 64
_G = 256
_CUT = 10.0
_CUT2 = _CUT * _CUT
_TB = 128
_WC = 512
_RB = 256
_PB = _RB * _K
_LOG2 = math.log(2.0)
_BIGV = 1e30
_BIGI = 1e9


def _ssp(v):
    return jnp.maximum(v, 0.0) + jnp.log1p(jnp.exp(-jnp.abs(v))) - _LOG2


def _nbr_body(pos_ref, posT_ref, batch_ref, batchT_ref, nbr_ref, d_ref):
    i = pl.program_id(0)
    tgx = posT_ref[0:1, :]
    tgy = posT_ref[1:2, :]
    tgz = posT_ref[2:3, :]
    tb = batchT_ref[0:1, :]
    b_lo = tb[:, 0:1]
    b_hi = tb[:, _TB - 1:_TB]
    tgid = jax.lax.broadcasted_iota(jnp.int32, (1, _TB), 1).astype(jnp.float32) + (i * _TB).astype(jnp.float32)

    allb = batch_ref[...]
    colf = jax.lax.broadcasted_iota(jnp.int32, (_N, 1), 0).astype(jnp.float32)
    w0f = jnp.min(jnp.where(allb == b_lo, colf, float(_N)))
    c1f = jnp.max(jnp.where(allb == b_hi, colf, -1.0)) + 1.0
    w0 = (w0f.astype(jnp.int32) // 8) * 8
    c1 = c1f.astype(jnp.int32)
    nchunks = (c1 - w0 + _WC - 1) // _WC

    def chunk(c, carry):
        bv, bi = carry
        start0 = w0 + c * _WC
        start = jnp.minimum(start0, _N - _WC)
        cand = pos_ref[pl.ds(start, _WC), :]
        cb = batch_ref[pl.ds(start, _WC), :]
        cidx = (jax.lax.broadcasted_iota(jnp.int32, (_WC, 1), 0).astype(jnp.float32)
                + start.astype(jnp.float32))
        dx = cand[:, 0:1] - tgx
        dy = cand[:, 1:2] - tgy
        dz = cand[:, 2:3] - tgz
        d2 = dx * dx + dy * dy + dz * dz
        valid = ((cb == tb) & (cidx != tgid) & (d2 < _CUT2)
                 & (cidx >= start0.astype(jnp.float32)))
        vals = jnp.where(valid, d2, _BIGV)
        cv = jnp.concatenate([vals, bv], axis=0)
        ci = jnp.concatenate([jnp.broadcast_to(cidx, (_WC, _TB)), bi], axis=0)
        nv, ni = [], []
        for _ in range(_K):
            m = jnp.min(cv, axis=0, keepdims=True)
            sel = jnp.min(jnp.where(cv == m, ci, _BIGI), axis=0, keepdims=True)
            kill = (cv == m) & (ci == sel)
            cv = jnp.where(kill, _BIGV, cv)
            ci = jnp.where(kill, _BIGI, ci)
            nv.append(m)
            ni.append(sel)
        return jnp.concatenate(nv, axis=0), jnp.concatenate(ni, axis=0)

    bv0 = jnp.full((_K, _TB), _BIGV, jnp.float32)
    bi0 = jnp.full((_K, _TB), _BIGI, jnp.float32)
    bv, bi = jax.lax.fori_loop(0, nchunks, chunk, (bv0, bi0))
    ok = bv < _CUT2
    d_ref[...] = jnp.where(ok, jnp.sqrt(jnp.maximum(bv, 1e-12)), 1e10)
    nbr_ref[...] = jnp.where(ok, bi, jnp.broadcast_to(tgid, (_K, _TB))).astype(jnp.int32)


def _nbr(pos, posT, batchf, batchfT):
    return pl.pallas_call(
        _nbr_body,
        grid=(_N // _TB,),
        in_specs=[
            pl.BlockSpec((_N, 3), lambda i: (0, 0)),
            pl.BlockSpec((3, _TB), lambda i: (0, i)),
            pl.BlockSpec((_N, 1), lambda i: (0, 0)),
            pl.BlockSpec((1, _TB), lambda i: (0, i)),
        ],
        out_specs=[
            pl.BlockSpec((_K, _TB), lambda i: (0, i)),
            pl.BlockSpec((_K, _TB), lambda i: (0, i)),
        ],
        out_shape=[
            jax.ShapeDtypeStruct((_K, _N), jnp.int32),
            jax.ShapeDtypeStruct((_K, _N), jnp.float32),
        ],
    )(pos, posT, batchf, batchfT)


def _embed_body(x_ref, emb_ref, o_ref):
    zf = x_ref[:, 5:6].astype(jnp.int32).astype(jnp.float32)
    oh = (zf == jax.lax.broadcasted_iota(jnp.int32, (1, _H), 1).astype(jnp.float32)).astype(jnp.float32)
    o_ref[...] = jnp.dot(oh, emb_ref[...], preferred_element_type=jnp.float32)


def _embed(x, emb_p):
    return pl.pallas_call(
        _embed_body,
        grid=(_N // _RB,),
        in_specs=[
            pl.BlockSpec((_RB, 11), lambda i: (i, 0)),
            pl.BlockSpec((_H, _H), lambda i: (0, 0)),
        ],
        out_specs=pl.BlockSpec((_RB, _H), lambda i: (i, 0)),
        out_shape=jax.ShapeDtypeStruct((_N, _H), jnp.float32),
    )(x, emb_p)


def _hx_body(h_ref, w_ref, o_ref):
    o_ref[...] = jnp.dot(h_ref[...], w_ref[...], preferred_element_type=jnp.float32)


def _hx(h, w):
    return pl.pallas_call(
        _hx_body,
        grid=(_N // _RB,),
        in_specs=[
            pl.BlockSpec((_RB, _H), lambda i: (i, 0)),
            pl.BlockSpec((_H, _H), lambda i: (0, 0)),
        ],
        out_specs=pl.BlockSpec((_RB, _H), lambda i: (i, 0)),
        out_shape=jax.ShapeDtypeStruct((_N, _H), jnp.float32),
    )(h, w)


_SC_NW = 32
_SC_BW = (_N * _K) // _SC_NW
_SC_CH = 128
_SC_NB = 4
_SC_GRP = _SC_CH * _SC_NB


def _sc_gather(table, idx2d):
    mesh = plsc.VectorSubcoreMesh(core_axis_name="c", subcore_axis_name="s")

    @functools.partial(
        pl.kernel,
        mesh=mesh,
        out_type=jax.ShapeDtypeStruct((_N * _K, _H), jnp.float32),
        scratch_types=[
            pltpu.VMEM((_SC_BW // _SC_CH, _SC_CH), jnp.int32),
            pltpu.VMEM((_SC_GRP, _H), jnp.float32),
            pltpu.SemaphoreType.DMA,
            pltpu.SemaphoreType.DMA,
        ],
    )
    def gather_k(table_hbm, idx_hbm, out_hbm, idx_v, rows_v, gsem, osem):
        wid = jax.lax.axis_index("s") * 2 + jax.lax.axis_index("c")
        wbase = wid * _SC_BW
        nch = _SC_BW // _SC_CH
        pltpu.sync_copy(idx_hbm.at[pl.ds(wid * nch, nch)], idx_v)

        def group(j, carry):
            gbase = j * _SC_GRP
            gets = []
            for b in range(_SC_NB):
                gets.append(pltpu.async_copy(
                    table_hbm.at[idx_v.at[j * _SC_NB + b]],
                    rows_v.at[pl.ds(b * _SC_CH, _SC_CH)], gsem))
            for g in gets:
                g.wait()
            pltpu.async_copy(rows_v, out_hbm.at[pl.ds(wbase + gbase, _SC_GRP)],
                             osem).wait()
            return carry

        jax.lax.fori_loop(0, _SC_BW // _SC_GRP, group, 0)

    return gather_k(table, idx2d)


def _msg_body(d_ref, g_ref, h_ref, offs_ref, coeff_ref, w1_ref, b1_ref,
              w2_ref, b2_ref, l2w_ref, l2b_ref, l3w_ref, l3b_ref, o_ref):
    d = d_ref[...]
    dd = d - offs_ref[...]
    e = jnp.exp(coeff_ref[0, 0] * (dd * dd))
    f1 = _ssp(jnp.dot(e, w1_ref[...], preferred_element_type=jnp.float32) + b1_ref[...])
    wf = jnp.dot(f1, w2_ref[...], preferred_element_type=jnp.float32) + b2_ref[...]
    dc = jnp.minimum(d, _CUT)
    cc = 0.5 * (jnp.cos(dc * (math.pi / _CUT)) + 1.0) * (d < _CUT).astype(jnp.float32)
    msg = wf * cc * g_ref[...]
    agg = jnp.sum(msg.reshape(_RB, _K, _H), axis=1)
    v = jnp.dot(agg, l2w_ref[...], preferred_element_type=jnp.float32) + l2b_ref[...]
    v = _ssp(v)
    v = jnp.dot(v, l3w_ref[...], preferred_element_type=jnp.float32) + l3b_ref[...]
    o_ref[...] = h_ref[...] + v


def _msg(d_flat, g, h, offs_p, coeff, w1p, b1, w2, b2, l2w, l2b, l3w, l3b):
    full = lambda a, b: pl.BlockSpec((a, b), lambda i: (0, 0))
    return pl.pallas_call(
        _msg_body,
        grid=(_N // _RB,),
        in_specs=[
            pl.BlockSpec((_PB, 1), lambda i: (i, 0)),
            pl.BlockSpec((_PB, _H), lambda i: (i, 0)),
            pl.BlockSpec((_RB, _H), lambda i: (i, 0)),
            full(1, _NGP), full(1, 1), full(_NGP, _H), full(1, _H),
            full(_H, _H), full(1, _H), full(_H, _H), full(1, _H),
            full(_H, _H), full(1, _H),
        ],
        out_specs=pl.BlockSpec((_RB, _H), lambda i: (i, 0)),
        out_shape=jax.ShapeDtypeStruct((_N, _H), jnp.float32),
    )(d_flat, g, h, offs_p, coeff, w1p, b1, w2, b2, l2w, l2b, l3w, l3b)


def _readout_body(h_ref, w1_ref, b1_ref, w2_ref, b2_ref, batch_ref, o_ref):
    t = _ssp(jnp.dot(h_ref[...], w1_ref[...], preferred_element_type=jnp.float32)
             + b1_ref[...])
    s = jnp.sum(t * w2_ref[...], axis=1, keepdims=True) + b2_ref[0, 0]
    gio = jax.lax.broadcasted_iota(jnp.int32, (1, _G), 1).astype(jnp.float32)
    m = (batch_ref[...] == gio).astype(jnp.float32)
    o_ref[...] = jnp.sum(m * s, axis=0, keepdims=True)


def _readout(h, w1, b1, w2, b2, batchf):
    full = lambda a, b: pl.BlockSpec((a, b), lambda i: (0, 0))
    return pl.pallas_call(
        _readout_body,
        grid=(1,),
        in_specs=[full(_N, _H), full(_H, 64), full(1, 64), full(1, 64),
                  full(1, 1), full(_N, 1)],
        out_specs=full(1, _G),
        out_shape=jax.ShapeDtypeStruct((1, _G), jnp.float32),
    )(h, w1, b1, w2, b2, batchf)


def kernel(x, pos, batch, emb, mlp_w1, mlp_b1, mlp_w2, mlp_b2, lin1_w, lin2_w,
           lin2_b, lin3_w, lin3_b, out_w1, out_b1, out_w2, out_b2):
    batchf = batch.astype(jnp.float32).reshape(_N, 1)
    batchfT = batchf.reshape(1, _N)
    posT = pos.T

    nbrT, dT = _nbr(pos, posT, batchf, batchfT)
    nbr_flat = nbrT.T.reshape(_N * _K // _SC_CH, _SC_CH)
    d_flat = dT.T.reshape(_N * _K, 1)

    emb_p = jnp.zeros((_H, _H), jnp.float32).at[:100].set(emb)
    h = _embed(x, emb_p)

    offs = jnp.linspace(0.0, _CUT, _NG)
    coeff = (-0.5 / (offs[1] - offs[0]) ** 2).reshape(1, 1).astype(jnp.float32)
    offs_p = jnp.concatenate([offs.astype(jnp.float32),
                              jnp.full((_NGP - _NG,), 1e6, jnp.float32)]).reshape(1, _NGP)
    w1p = jnp.pad(mlp_w1, ((0, 0), (0, _NGP - _NG), (0, 0)))

    for i in range(6):
        hx = _hx(h, lin1_w[i])
        g = _sc_gather(hx, nbr_flat)
        h = _msg(d_flat, g, h, offs_p, coeff, w1p[i], mlp_b1[i].reshape(1, _H),
                 mlp_w2[i], mlp_b2[i].reshape(1, _H), lin2_w[i],
                 lin2_b[i].reshape(1, _H), lin3_w[i], lin3_b[i].reshape(1, _H))

    out = _readout(h, out_w1, out_b1.reshape(1, 64), out_w2.reshape(1, 64),
                   out_b2.reshape(1, 1), batchf)
    return out.reshape(-1)


# speedup vs baseline: 6.5949x; 1.0498x over previous
import functools
import math

import jax
import jax.numpy as jnp
from jax.experimental import pallas as pl
from jax.experimental.pallas import tpu as pltpu
from jax.experimental.pallas import tpu_sc as plsc

_N = 4096
_K = 32
_H = 128
_NG = 50
_NGP = 64
_G = 256
_CUT = 10.0
_CUT2 = _CUT * _CUT
_TB = 128
_WC = 256
_RB = 256
_PB = _RB * _K
_LOG2 = math.log(2.0)
_BIGV = 1e30
_BIGI = 1e9


def _ssp(v):
    return jnp.maximum(v, 0.0) + jnp.log1p(jnp.exp(-jnp.abs(v))) - _LOG2


def _nbr_body(pos_ref, posT_ref, batch_ref, batchT_ref, nbr_ref, d_ref):
    i = pl.program_id(0)
    tgx = posT_ref[0:1, :]
    tgy = posT_ref[1:2, :]
    tgz = posT_ref[2:3, :]
    tb = batchT_ref[0:1, :]
    b_lo = tb[:, 0:1]
    b_hi = tb[:, _TB - 1:_TB]
    tgid = jax.lax.broadcasted_iota(jnp.int32, (1, _TB), 1).astype(jnp.float32) + (i * _TB).astype(jnp.float32)

    allb = batch_ref[...]
    colf = jax.lax.broadcasted_iota(jnp.int32, (_N, 1), 0).astype(jnp.float32)
    w0f = jnp.min(jnp.where(allb == b_lo, colf, float(_N)))
    c1f = jnp.max(jnp.where(allb == b_hi, colf, -1.0)) + 1.0
    w0 = (w0f.astype(jnp.int32) // 8) * 8
    c1 = c1f.astype(jnp.int32)
    nchunks = (c1 - w0 + _WC - 1) // _WC

    def chunk(c, carry):
        bv, bi = carry
        start0 = w0 + c * _WC
        start = jnp.minimum(start0, _N - _WC)
        cand = pos_ref[pl.ds(start, _WC), :]
        cb = batch_ref[pl.ds(start, _WC), :]
        cidx = (jax.lax.broadcasted_iota(jnp.int32, (_WC, 1), 0).astype(jnp.float32)
                + start.astype(jnp.float32))
        dx = cand[:, 0:1] - tgx
        dy = cand[:, 1:2] - tgy
        dz = cand[:, 2:3] - tgz
        d2 = dx * dx + dy * dy + dz * dz
        valid = ((cb == tb) & (cidx != tgid) & (d2 < _CUT2)
                 & (cidx >= start0.astype(jnp.float32)))
        vals = jnp.where(valid, d2, _BIGV)
        cv = jnp.concatenate([vals, bv], axis=0)
        ci = jnp.concatenate([jnp.broadcast_to(cidx, (_WC, _TB)), bi], axis=0)
        nv, ni = [], []
        for _ in range(_K):
            m = jnp.min(cv, axis=0, keepdims=True)
            sel = jnp.min(jnp.where(cv == m, ci, _BIGI), axis=0, keepdims=True)
            kill = (cv == m) & (ci == sel)
            cv = jnp.where(kill, _BIGV, cv)
            ci = jnp.where(kill, _BIGI, ci)
            nv.append(m)
            ni.append(sel)
        return jnp.concatenate(nv, axis=0), jnp.concatenate(ni, axis=0)

    bv0 = jnp.full((_K, _TB), _BIGV, jnp.float32)
    bi0 = jnp.full((_K, _TB), _BIGI, jnp.float32)
    bv, bi = jax.lax.fori_loop(0, nchunks, chunk, (bv0, bi0))
    ok = bv < _CUT2
    d_ref[...] = jnp.where(ok, jnp.sqrt(jnp.maximum(bv, 1e-12)), 1e10)
    nbr_ref[...] = jnp.where(ok, bi, jnp.broadcast_to(tgid, (_K, _TB))).astype(jnp.int32)


def _nbr(pos, posT, batchf, batchfT):
    return pl.pallas_call(
        _nbr_body,
        grid=(_N // _TB,),
        in_specs=[
            pl.BlockSpec((_N, 3), lambda i: (0, 0)),
            pl.BlockSpec((3, _TB), lambda i: (0, i)),
            pl.BlockSpec((_N, 1), lambda i: (0, 0)),
            pl.BlockSpec((1, _TB), lambda i: (0, i)),
        ],
        out_specs=[
            pl.BlockSpec((_K, _TB), lambda i: (0, i)),
            pl.BlockSpec((_K, _TB), lambda i: (0, i)),
        ],
        out_shape=[
            jax.ShapeDtypeStruct((_K, _N), jnp.int32),
            jax.ShapeDtypeStruct((_K, _N), jnp.float32),
        ],
    )(pos, posT, batchf, batchfT)


def _embed_body(x_ref, emb_ref, w_ref, o_ref, ox_ref):
    zf = x_ref[:, 5:6].astype(jnp.int32).astype(jnp.float32)
    oh = (zf == jax.lax.broadcasted_iota(jnp.int32, (1, _H), 1).astype(jnp.float32)).astype(jnp.float32)
    h0 = jnp.dot(oh, emb_ref[...], preferred_element_type=jnp.float32)
    o_ref[...] = h0
    ox_ref[...] = jnp.dot(h0, w_ref[...], preferred_element_type=jnp.float32)


def _embed(x, emb_p, w0):
    return pl.pallas_call(
        _embed_body,
        grid=(_N // _RB,),
        in_specs=[
            pl.BlockSpec((_RB, 11), lambda i: (i, 0)),
            pl.BlockSpec((_H, _H), lambda i: (0, 0)),
            pl.BlockSpec((_H, _H), lambda i: (0, 0)),
        ],
        out_specs=[pl.BlockSpec((_RB, _H), lambda i: (i, 0)),
                   pl.BlockSpec((_RB, _H), lambda i: (i, 0))],
        out_shape=[jax.ShapeDtypeStruct((_N, _H), jnp.float32),
                   jax.ShapeDtypeStruct((_N, _H), jnp.float32)],
    )(x, emb_p, w0)


_SC_NW = 32
_SC_BW = (_N * _K) // _SC_NW
_SC_CH = 128
_SC_NB = 4
_SC_GRP = _SC_CH * _SC_NB


def _sc_gather(table, idx2d):
    mesh = plsc.VectorSubcoreMesh(core_axis_name="c", subcore_axis_name="s")

    @functools.partial(
        pl.kernel,
        mesh=mesh,
        out_type=jax.ShapeDtypeStruct((_N * _K, _H), jnp.float32),
        scratch_types=[
            pltpu.VMEM((_SC_BW // _SC_CH, _SC_CH), jnp.int32),
            pltpu.VMEM((_SC_GRP, _H), jnp.float32),
            pltpu.SemaphoreType.DMA,
            pltpu.SemaphoreType.DMA,
        ],
    )
    def gather_k(table_hbm, idx_hbm, out_hbm, idx_v, rows_v, gsem, osem):
        wid = jax.lax.axis_index("s") * 2 + jax.lax.axis_index("c")
        wbase = wid * _SC_BW
        nch = _SC_BW // _SC_CH
        pltpu.sync_copy(idx_hbm.at[pl.ds(wid * nch, nch)], idx_v)

        def group(j, carry):
            gbase = j * _SC_GRP
            gets = []
            for b in range(_SC_NB):
                gets.append(pltpu.async_copy(
                    table_hbm.at[idx_v.at[j * _SC_NB + b]],
                    rows_v.at[pl.ds(b * _SC_CH, _SC_CH)], gsem))
            for g in gets:
                g.wait()
            pltpu.async_copy(rows_v, out_hbm.at[pl.ds(wbase + gbase, _SC_GRP)],
                             osem).wait()
            return carry

        jax.lax.fori_loop(0, _SC_BW // _SC_GRP, group, 0)

    return gather_k(table, idx2d)


def _msg_body(d_ref, g_ref, h_ref, offs_ref, coeff_ref, w1_ref, b1_ref,
              w2_ref, b2_ref, l2w_ref, l2b_ref, l3w_ref, l3b_ref, lnx_ref,
              o_ref, ox_ref):
    d = d_ref[...]
    dd = d - offs_ref[...]
    e = jnp.exp(coeff_ref[0, 0] * (dd * dd))
    f1 = _ssp(jnp.dot(e, w1_ref[...], preferred_element_type=jnp.float32) + b1_ref[...])
    wf = jnp.dot(f1, w2_ref[...], preferred_element_type=jnp.float32) + b2_ref[...]
    dc = jnp.minimum(d, _CUT)
    cc = 0.5 * (jnp.cos(dc * (math.pi / _CUT)) + 1.0) * (d < _CUT).astype(jnp.float32)
    msg = wf * cc * g_ref[...]
    agg = jnp.sum(msg.reshape(_RB, _K, _H), axis=1)
    v = jnp.dot(agg, l2w_ref[...], preferred_element_type=jnp.float32) + l2b_ref[...]
    v = _ssp(v)
    v = jnp.dot(v, l3w_ref[...], preferred_element_type=jnp.float32) + l3b_ref[...]
    hn = h_ref[...] + v
    o_ref[...] = hn
    ox_ref[...] = jnp.dot(hn, lnx_ref[...], preferred_element_type=jnp.float32)


def _msg(d_flat, g, h, offs_p, coeff, w1p, b1, w2, b2, l2w, l2b, l3w, l3b, lnx):
    full = lambda a, b: pl.BlockSpec((a, b), lambda i: (0, 0))
    return pl.pallas_call(
        _msg_body,
        grid=(_N // _RB,),
        in_specs=[
            pl.BlockSpec((_PB, 1), lambda i: (i, 0)),
            pl.BlockSpec((_PB, _H), lambda i: (i, 0)),
            pl.BlockSpec((_RB, _H), lambda i: (i, 0)),
            full(1, _NGP), full(1, 1), full(_NGP, _H), full(1, _H),
            full(_H, _H), full(1, _H), full(_H, _H), full(1, _H),
            full(_H, _H), full(1, _H), full(_H, _H),
        ],
        out_specs=[pl.BlockSpec((_RB, _H), lambda i: (i, 0)),
                   pl.BlockSpec((_RB, _H), lambda i: (i, 0))],
        out_shape=[jax.ShapeDtypeStruct((_N, _H), jnp.float32),
                   jax.ShapeDtypeStruct((_N, _H), jnp.float32)],
    )(d_flat, g, h, offs_p, coeff, w1p, b1, w2, b2, l2w, l2b, l3w, l3b, lnx)


def _readout_body(h_ref, w1_ref, b1_ref, w2_ref, b2_ref, batch_ref, o_ref):
    t = _ssp(jnp.dot(h_ref[...], w1_ref[...], preferred_element_type=jnp.float32)
             + b1_ref[...])
    s = jnp.sum(t * w2_ref[...], axis=1, keepdims=True) + b2_ref[0, 0]
    gio = jax.lax.broadcasted_iota(jnp.int32, (1, _G), 1).astype(jnp.float32)
    m = (batch_ref[...] == gio).astype(jnp.float32)
    o_ref[...] = jnp.sum(m * s, axis=0, keepdims=True)


def _readout(h, w1, b1, w2, b2, batchf):
    full = lambda a, b: pl.BlockSpec((a, b), lambda i: (0, 0))
    return pl.pallas_call(
        _readout_body,
        grid=(1,),
        in_specs=[full(_N, _H), full(_H, 64), full(1, 64), full(1, 64),
                  full(1, 1), full(_N, 1)],
        out_specs=full(1, _G),
        out_shape=jax.ShapeDtypeStruct((1, _G), jnp.float32),
    )(h, w1, b1, w2, b2, batchf)


def kernel(x, pos, batch, emb, mlp_w1, mlp_b1, mlp_w2, mlp_b2, lin1_w, lin2_w,
           lin2_b, lin3_w, lin3_b, out_w1, out_b1, out_w2, out_b2):
    batchf = batch.astype(jnp.float32).reshape(_N, 1)
    batchfT = batchf.reshape(1, _N)
    posT = pos.T

    nbrT, dT = _nbr(pos, posT, batchf, batchfT)
    nbr_flat = nbrT.T.reshape(_N * _K // _SC_CH, _SC_CH)
    d_flat = dT.T.reshape(_N * _K, 1)

    emb_p = jnp.zeros((_H, _H), jnp.float32).at[:100].set(emb)
    h, hx = _embed(x, emb_p, lin1_w[0])

    offs = jnp.linspace(0.0, _CUT, _NG)
    coeff = (-0.5 / (offs[1] - offs[0]) ** 2).reshape(1, 1).astype(jnp.float32)
    offs_p = jnp.concatenate([offs.astype(jnp.float32),
                              jnp.full((_NGP - _NG,), 1e6, jnp.float32)]).reshape(1, _NGP)
    w1p = jnp.pad(mlp_w1, ((0, 0), (0, _NGP - _NG), (0, 0)))

    for i in range(6):
        g = _sc_gather(hx, nbr_flat)
        h, hx = _msg(d_flat, g, h, offs_p, coeff, w1p[i], mlp_b1[i].reshape(1, _H),
                     mlp_w2[i], mlp_b2[i].reshape(1, _H), lin2_w[i],
                     lin2_b[i].reshape(1, _H), lin3_w[i], lin3_b[i].reshape(1, _H),
                     lin1_w[(i + 1) % 6])

    out = _readout(h, out_w1, out_b1.reshape(1, 64), out_w2.reshape(1, 64),
                   out_b2.reshape(1, 1), batchf)
    return out.reshape(-1)


# speedup vs baseline: 6.6569x; 1.0094x over previous
import functools
import math

import jax
import jax.numpy as jnp
from jax.experimental import pallas as pl
from jax.experimental.pallas import tpu as pltpu
from jax.experimental.pallas import tpu_sc as plsc

_N = 4096
_K = 32
_H = 128
_NG = 50
_NGP = 64
_G = 256
_CUT = 10.0
_CUT2 = _CUT * _CUT
_TB = 128
_WC = 256
_RB = 256
_PB = _RB * _K
_LOG2 = math.log(2.0)
_BIGV = 1e30
_BIGI = 1e9


def _ssp(v):
    return jnp.maximum(v, 0.0) + jnp.log1p(jnp.exp(-jnp.abs(v))) - _LOG2


def _nbr_body(pos_ref, posT_ref, batch_ref, batchT_ref, nbr_ref, d_ref):
    i = pl.program_id(0)
    tgx = posT_ref[0:1, :]
    tgy = posT_ref[1:2, :]
    tgz = posT_ref[2:3, :]
    tb = batchT_ref[0:1, :]
    b_lo = tb[:, 0:1]
    b_hi = tb[:, _TB - 1:_TB]
    tgid = jax.lax.broadcasted_iota(jnp.int32, (1, _TB), 1).astype(jnp.float32) + (i * _TB).astype(jnp.float32)

    allb = batch_ref[...]
    colf = jax.lax.broadcasted_iota(jnp.int32, (_N, 1), 0).astype(jnp.float32)
    w0f = jnp.min(jnp.where(allb == b_lo, colf, float(_N)))
    c1f = jnp.max(jnp.where(allb == b_hi, colf, -1.0)) + 1.0
    w0 = (w0f.astype(jnp.int32) // 8) * 8
    c1 = c1f.astype(jnp.int32)
    nchunks = (c1 - w0 + _WC - 1) // _WC

    def chunk(c, carry):
        bv, bi = carry
        start0 = w0 + c * _WC
        start = jnp.minimum(start0, _N - _WC)
        cand = pos_ref[pl.ds(start, _WC), :]
        cb = batch_ref[pl.ds(start, _WC), :]
        cidx = (jax.lax.broadcasted_iota(jnp.int32, (_WC, 1), 0).astype(jnp.float32)
                + start.astype(jnp.float32))
        dx = cand[:, 0:1] - tgx
        dy = cand[:, 1:2] - tgy
        dz = cand[:, 2:3] - tgz
        d2 = dx * dx + dy * dy + dz * dz
        valid = ((cb == tb) & (cidx != tgid) & (d2 < _CUT2)
                 & (cidx >= start0.astype(jnp.float32)))
        vals = jnp.where(valid, d2, _BIGV)
        cv = jnp.concatenate([vals, bv], axis=0)
        ci = jnp.concatenate([jnp.broadcast_to(cidx, (_WC, _TB)), bi], axis=0)
        nv, ni = [], []
        for _ in range(_K):
            m = jnp.min(cv, axis=0, keepdims=True)
            sel = jnp.min(jnp.where(cv == m, ci, _BIGI), axis=0, keepdims=True)
            kill = (cv == m) & (ci == sel)
            cv = jnp.where(kill, _BIGV, cv)
            ci = jnp.where(kill, _BIGI, ci)
            nv.append(m)
            ni.append(sel)
        return jnp.concatenate(nv, axis=0), jnp.concatenate(ni, axis=0)

    bv0 = jnp.full((_K, _TB), _BIGV, jnp.float32)
    bi0 = jnp.full((_K, _TB), _BIGI, jnp.float32)
    bv, bi = jax.lax.fori_loop(0, nchunks, chunk, (bv0, bi0))
    ok = bv < _CUT2
    d_ref[...] = jnp.where(ok, jnp.sqrt(jnp.maximum(bv, 1e-12)), 1e10)
    nbr_ref[...] = jnp.where(ok, bi, jnp.broadcast_to(tgid, (_K, _TB))).astype(jnp.int32)


def _nbr(pos, posT, batchf, batchfT):
    return pl.pallas_call(
        _nbr_body,
        grid=(_N // _TB,),
        in_specs=[
            pl.BlockSpec((_N, 3), lambda i: (0, 0)),
            pl.BlockSpec((3, _TB), lambda i: (0, i)),
            pl.BlockSpec((_N, 1), lambda i: (0, 0)),
            pl.BlockSpec((1, _TB), lambda i: (0, i)),
        ],
        out_specs=[
            pl.BlockSpec((_K, _TB), lambda i: (0, i)),
            pl.BlockSpec((_K, _TB), lambda i: (0, i)),
        ],
        out_shape=[
            jax.ShapeDtypeStruct((_K, _N), jnp.int32),
            jax.ShapeDtypeStruct((_K, _N), jnp.float32),
        ],
    )(pos, posT, batchf, batchfT)


def _embed_body(x_ref, emb_ref, w_ref, o_ref, ox_ref):
    zf = x_ref[:, 5:6].astype(jnp.int32).astype(jnp.float32)
    oh = (zf == jax.lax.broadcasted_iota(jnp.int32, (1, _H), 1).astype(jnp.float32)).astype(jnp.float32)
    h0 = jnp.dot(oh, emb_ref[...], preferred_element_type=jnp.float32)
    o_ref[...] = h0
    ox_ref[...] = jnp.dot(h0, w_ref[...], preferred_element_type=jnp.float32)


def _embed(x, emb_p, w0):
    return pl.pallas_call(
        _embed_body,
        grid=(_N // _RB,),
        in_specs=[
            pl.BlockSpec((_RB, 11), lambda i: (i, 0)),
            pl.BlockSpec((_H, _H), lambda i: (0, 0)),
            pl.BlockSpec((_H, _H), lambda i: (0, 0)),
        ],
        out_specs=[pl.BlockSpec((_RB, _H), lambda i: (i, 0)),
                   pl.BlockSpec((_RB, _H), lambda i: (i, 0))],
        out_shape=[jax.ShapeDtypeStruct((_N, _H), jnp.float32),
                   jax.ShapeDtypeStruct((_N, _H), jnp.float32)],
    )(x, emb_p, w0)


_SC_NW = 32
_SC_BW = (_N * _K) // _SC_NW
_SC_CH = 128
_SC_NB = 4
_SC_GRP = _SC_CH * _SC_NB


def _sc_gather(table, idx2d):
    mesh = plsc.VectorSubcoreMesh(core_axis_name="c", subcore_axis_name="s")

    @functools.partial(
        pl.kernel,
        mesh=mesh,
        out_type=jax.ShapeDtypeStruct((_N * _K, _H), jnp.float32),
        scratch_types=[
            pltpu.VMEM((_SC_BW // _SC_CH, _SC_CH), jnp.int32),
            pltpu.VMEM((_SC_GRP, _H), jnp.float32),
            pltpu.SemaphoreType.DMA,
            pltpu.SemaphoreType.DMA,
        ],
    )
    def gather_k(table_hbm, idx_hbm, out_hbm, idx_v, rows_v, gsem, osem):
        wid = jax.lax.axis_index("s") * 2 + jax.lax.axis_index("c")
        wbase = wid * _SC_BW
        nch = _SC_BW // _SC_CH
        pltpu.sync_copy(idx_hbm.at[pl.ds(wid * nch, nch)], idx_v)

        def group(j, carry):
            gbase = j * _SC_GRP
            gets = []
            for b in range(_SC_NB):
                gets.append(pltpu.async_copy(
                    table_hbm.at[idx_v.at[j * _SC_NB + b]],
                    rows_v.at[pl.ds(b * _SC_CH, _SC_CH)], gsem))
            for g in gets:
                g.wait()
            pltpu.async_copy(rows_v, out_hbm.at[pl.ds(wbase + gbase, _SC_GRP)],
                             osem).wait()
            return carry

        jax.lax.fori_loop(0, _SC_BW // _SC_GRP, group, 0)

    return gather_k(table, idx2d)


def _msg_body(d_ref, g_ref, h_ref, offs_ref, coeff_ref, w1_ref, b1_ref,
              w2_ref, b2_ref, l2w_ref, l2b_ref, l3w_ref, l3b_ref, lnx_ref,
              o_ref, ox_ref):
    d3 = d_ref[...][:, :, None]
    dd = d3 - offs_ref[...][None, :, :]
    e = jnp.exp(coeff_ref[0, 0] * (dd * dd)).reshape(_PB, _NGP)
    f1 = _ssp(jnp.dot(e, w1_ref[...], preferred_element_type=jnp.float32) + b1_ref[...])
    wf = jnp.dot(f1, w2_ref[...], preferred_element_type=jnp.float32) + b2_ref[...]
    dc = jnp.minimum(d3, _CUT)
    cc = 0.5 * (jnp.cos(dc * (math.pi / _CUT)) + 1.0) * (d3 < _CUT).astype(jnp.float32)
    msg = wf.reshape(_RB, _K, _H) * cc * g_ref[...]
    agg = jnp.sum(msg, axis=1)
    v = jnp.dot(agg, l2w_ref[...], preferred_element_type=jnp.float32) + l2b_ref[...]
    v = _ssp(v)
    v = jnp.dot(v, l3w_ref[...], preferred_element_type=jnp.float32) + l3b_ref[...]
    hn = h_ref[...] + v
    o_ref[...] = hn
    ox_ref[...] = jnp.dot(hn, lnx_ref[...], preferred_element_type=jnp.float32)


def _msg(d_flat, g, h, offs_p, coeff, w1p, b1, w2, b2, l2w, l2b, l3w, l3b, lnx):
    full = lambda a, b: pl.BlockSpec((a, b), lambda i: (0, 0))
    return pl.pallas_call(
        _msg_body,
        grid=(_N // _RB,),
        in_specs=[
            pl.BlockSpec((_RB, _K), lambda i: (i, 0)),
            pl.BlockSpec((_RB, _K, _H), lambda i: (i, 0, 0)),
            pl.BlockSpec((_RB, _H), lambda i: (i, 0)),
            full(1, _NGP), full(1, 1), full(_NGP, _H), full(1, _H),
            full(_H, _H), full(1, _H), full(_H, _H), full(1, _H),
            full(_H, _H), full(1, _H), full(_H, _H),
        ],
        out_specs=[pl.BlockSpec((_RB, _H), lambda i: (i, 0)),
                   pl.BlockSpec((_RB, _H), lambda i: (i, 0))],
        out_shape=[jax.ShapeDtypeStruct((_N, _H), jnp.float32),
                   jax.ShapeDtypeStruct((_N, _H), jnp.float32)],
    )(d_flat, g, h, offs_p, coeff, w1p, b1, w2, b2, l2w, l2b, l3w, l3b, lnx)


def _readout_body(h_ref, w1_ref, b1_ref, w2_ref, b2_ref, batch_ref, o_ref):
    t = _ssp(jnp.dot(h_ref[...], w1_ref[...], preferred_element_type=jnp.float32)
             + b1_ref[...])
    s = jnp.sum(t * w2_ref[...], axis=1, keepdims=True) + b2_ref[0, 0]
    gio = jax.lax.broadcasted_iota(jnp.int32, (1, _G), 1).astype(jnp.float32)
    m = (batch_ref[...] == gio).astype(jnp.float32)
    o_ref[...] = jnp.sum(m * s, axis=0, keepdims=True)


def _readout(h, w1, b1, w2, b2, batchf):
    full = lambda a, b: pl.BlockSpec((a, b), lambda i: (0, 0))
    return pl.pallas_call(
        _readout_body,
        grid=(1,),
        in_specs=[full(_N, _H), full(_H, 64), full(1, 64), full(1, 64),
                  full(1, 1), full(_N, 1)],
        out_specs=full(1, _G),
        out_shape=jax.ShapeDtypeStruct((1, _G), jnp.float32),
    )(h, w1, b1, w2, b2, batchf)


def kernel(x, pos, batch, emb, mlp_w1, mlp_b1, mlp_w2, mlp_b2, lin1_w, lin2_w,
           lin2_b, lin3_w, lin3_b, out_w1, out_b1, out_w2, out_b2):
    batchf = batch.astype(jnp.float32).reshape(_N, 1)
    batchfT = batchf.reshape(1, _N)
    posT = pos.T

    nbrT, dT = _nbr(pos, posT, batchf, batchfT)
    nbr_flat = nbrT.T.reshape(_N * _K // _SC_CH, _SC_CH)
    d_nk = dT.T

    emb_p = jnp.zeros((_H, _H), jnp.float32).at[:100].set(emb)
    h, hx = _embed(x, emb_p, lin1_w[0])

    offs = jnp.linspace(0.0, _CUT, _NG)
    coeff = (-0.5 / (offs[1] - offs[0]) ** 2).reshape(1, 1).astype(jnp.float32)
    offs_p = jnp.concatenate([offs.astype(jnp.float32),
                              jnp.full((_NGP - _NG,), 1e6, jnp.float32)]).reshape(1, _NGP)
    w1p = jnp.pad(mlp_w1, ((0, 0), (0, _NGP - _NG), (0, 0)))

    for i in range(6):
        g = _sc_gather(hx, nbr_flat).reshape(_N, _K, _H)
        h, hx = _msg(d_nk, g, h, offs_p, coeff, w1p[i], mlp_b1[i].reshape(1, _H),
                     mlp_w2[i], mlp_b2[i].reshape(1, _H), lin2_w[i],
                     lin2_b[i].reshape(1, _H), lin3_w[i], lin3_b[i].reshape(1, _H),
                     lin1_w[(i + 1) % 6])

    out = _readout(h, out_w1, out_b1.reshape(1, 64), out_w2.reshape(1, 64),
                   out_b2.reshape(1, 1), batchf)
    return out.reshape(-1)
